# baseline (speedup 1.0000x reference)
import functools

import jax
import jax.numpy as jnp
from jax import lax
from jax.experimental import pallas as pl
from jax.experimental.pallas import tpu as pltpu
from jax.experimental.pallas import tpu_sc as plsc

_B = 16384
_D = 128
_K = 38
_ROWS = 4096
_NW = 32
_RPW = _B // _NW
_UNROLL = 4


def _mlp_body(feat_ref, w1_ref, b1_ref, w2_ref, b2_ref, w3_ref, b3_ref, out_ref):
    xt = feat_ref[:].T
    h = jnp.maximum(
        jnp.dot(w1_ref[:], xt, preferred_element_type=jnp.float32) + b1_ref[:], 0.0
    )
    h = jnp.dot(w2_ref[:], h, preferred_element_type=jnp.float32) + b2_ref[:]
    t = jnp.dot(w3_ref[:], h, preferred_element_type=jnp.float32) + b3_ref[:]
    out_ref[:] = (1.0 / (1.0 + jnp.exp(-t))).T


def _tc_mlp(feature, W1, bb1, W2, bb2, W3, bb3):
    grid = _B // _ROWS
    row_spec = pl.BlockSpec((_ROWS, _D), lambda i: (i, 0))
    w_spec = pl.BlockSpec((_D, _D), lambda i: (0, 0))
    b_spec = pl.BlockSpec((_D, 1), lambda i: (0, 0))
    return pl.pallas_call(
        _mlp_body,
        grid=(grid,),
        in_specs=[row_spec, w_spec, b_spec, w_spec, b_spec, w_spec, b_spec],
        out_specs=row_spec,
        out_shape=jax.ShapeDtypeStruct((_B, _D), jnp.float32),
    )(feature, W1, bb1, W2, bb2, W3, bb3)


def _sc_mask_body(x2_hbm, out_hbm, buf, sem):
    wid = lax.axis_index("s") * 2 + lax.axis_index("c")
    base = wid * _RPW
    pltpu.async_copy(x2_hbm.at[pl.ds(base, _RPW)], buf, sem).wait()

    iota = lax.iota(jnp.int32, 16)
    perms = [iota ^ (1 << p) for p in range(4)]
    _dn = lax.GatherDimensionNumbers(
        offset_dims=(), collapsed_slice_dims=(0,), start_index_map=(0,)
    )

    def _lane_sum(c):
        for p in perms:
            c = c + lax.gather(
                c, p[:, None], _dn, (1,),
                mode=lax.GatherScatterMode.PROMISE_IN_BOUNDS,
            )
        return c

    def row_body(g, carry):
        for u in range(_UNROLL):
            r = g * _UNROLL + u
            vals = [buf[r, pl.ds(16 * j, 16)] for j in range(8)]
            xis = [lax.bitcast_convert_type(v, jnp.int32) for v in vals]
            lo = jnp.zeros((16,), jnp.int32)
            hi = jnp.full((16,), 0x3F800000, jnp.int32)
            for _ in range(30):
                mid = lax.shift_right_logical(lo + hi, 1)
                acc = jnp.where(xis[0] <= mid, 1, 0)
                for j in range(1, 8):
                    acc = acc + jnp.where(xis[j] <= mid, 1, 0)
                c = _lane_sum(acc)
                geq = c >= _K
                hi = jnp.where(geq, mid, hi)
                lo = jnp.where(geq, lo, mid + 1)
            for j in range(8):
                buf[r, pl.ds(16 * j, 16)] = jnp.where(xis[j] > lo, vals[j], 0.0)
        return carry

    lax.fori_loop(0, _RPW // _UNROLL, row_body, jnp.int32(0))
    pltpu.async_copy(buf, out_hbm.at[pl.ds(base, _RPW)], sem).wait()


_sc_mask = functools.partial(
    pl.kernel,
    out_type=jax.ShapeDtypeStruct((_B, _D), jnp.float32),
    mesh=plsc.VectorSubcoreMesh(core_axis_name="c", subcore_axis_name="s"),
    scratch_types=[
        pltpu.VMEM((_RPW, _D), jnp.float32),
        pltpu.SemaphoreType.DMA,
    ],
)(_sc_mask_body)


@jax.jit
def kernel(feature, W1, b1, W2, b2, W3, b3):
    bb1 = b1.reshape(_D, 1)
    bb2 = b2.reshape(_D, 1)
    bb3 = b3.reshape(_D, 1)
    x2 = _tc_mlp(feature, W1, bb1, W2, bb2, W3, bb3)
    return _sc_mask(x2)


# speedup vs baseline: 1.8156x; 1.8156x over previous
import functools

import jax
import jax.numpy as jnp
from jax import lax
from jax.experimental import pallas as pl
from jax.experimental.pallas import tpu as pltpu
from jax.experimental.pallas import tpu_sc as plsc

_B = 16384
_D = 128
_K = 38
_ROWS = 4096
_NW = 32
_RPW = _B // _NW
_UNROLL = 2


def _mlp_body(feat_ref, w1_ref, b1_ref, w2_ref, b2_ref, w3_ref, b3_ref, out_ref):
    xt = feat_ref[:].T
    h = jnp.maximum(
        jnp.dot(w1_ref[:], xt, preferred_element_type=jnp.float32) + b1_ref[:], 0.0
    )
    h = jnp.dot(w2_ref[:], h, preferred_element_type=jnp.float32) + b2_ref[:]
    t = jnp.dot(w3_ref[:], h, preferred_element_type=jnp.float32) + b3_ref[:]
    out_ref[:] = (1.0 / (1.0 + jnp.exp(-t))).T


def _tc_mlp(feature, W1, bb1, W2, bb2, W3, bb3):
    grid = _B // _ROWS
    row_spec = pl.BlockSpec((_ROWS, _D), lambda i: (i, 0))
    w_spec = pl.BlockSpec((_D, _D), lambda i: (0, 0))
    b_spec = pl.BlockSpec((_D, 1), lambda i: (0, 0))
    return pl.pallas_call(
        _mlp_body,
        grid=(grid,),
        in_specs=[row_spec, w_spec, b_spec, w_spec, b_spec, w_spec, b_spec],
        out_specs=row_spec,
        out_shape=jax.ShapeDtypeStruct((_B, _D), jnp.float32),
    )(feature, W1, bb1, W2, bb2, W3, bb3)


def _sc_mask_body(x2_hbm, out_hbm, buf, sem):
    wid = lax.axis_index("s") * 2 + lax.axis_index("c")
    base = wid * _RPW
    pltpu.async_copy(x2_hbm.at[pl.ds(base, _RPW)], buf, sem).wait()

    iota = lax.iota(jnp.int32, 16)
    perms = [iota ^ (1 << p) for p in range(4)]
    _dn = lax.GatherDimensionNumbers(
        offset_dims=(), collapsed_slice_dims=(0,), start_index_map=(0,)
    )

    def _lane_sum(c):
        for p in perms:
            c = c + lax.gather(
                c, p[:, None], _dn, (1,),
                mode=lax.GatherScatterMode.PROMISE_IN_BOUNDS,
            )
        return c

    def row_body(g, carry):
        for u in range(_UNROLL):
            r = g * _UNROLL + u
            vals = [buf[r, pl.ds(16 * j, 16)] for j in range(8)]
            xis = [lax.bitcast_convert_type(v, jnp.int32) for v in vals]
            lo = jnp.zeros((16,), jnp.int32)
            hi = jnp.full((16,), 0x3F800000, jnp.int32)
            for _ in range(30):
                mid = lax.shift_right_logical(lo + hi, 1)
                acc = jnp.where(xis[0] <= mid, 1, 0)
                for j in range(1, 8):
                    acc = acc + jnp.where(xis[j] <= mid, 1, 0)
                c = _lane_sum(acc)
                geq = c >= _K
                hi = jnp.where(geq, mid, hi)
                lo = jnp.where(geq, lo, mid + 1)
            for j in range(8):
                buf[r, pl.ds(16 * j, 16)] = jnp.where(xis[j] > lo, vals[j], 0.0)
        return carry

    lax.fori_loop(0, _RPW // _UNROLL, row_body, jnp.int32(0))
    pltpu.async_copy(buf, out_hbm.at[pl.ds(base, _RPW)], sem).wait()


_sc_mask = functools.partial(
    pl.kernel,
    out_type=jax.ShapeDtypeStruct((_B, _D), jnp.float32),
    mesh=plsc.VectorSubcoreMesh(core_axis_name="c", subcore_axis_name="s"),
    scratch_types=[
        pltpu.VMEM((_RPW, _D), jnp.float32),
        pltpu.SemaphoreType.DMA,
    ],
)(_sc_mask_body)


@jax.jit
def kernel(feature, W1, b1, W2, b2, W3, b3):
    bb1 = b1.reshape(_D, 1)
    bb2 = b2.reshape(_D, 1)
    bb3 = b3.reshape(_D, 1)
    x2 = _tc_mlp(feature, W1, bb1, W2, bb2, W3, bb3)
    return _sc_mask(x2)


# speedup vs baseline: 4.3951x; 2.4207x over previous
import functools

import jax
import jax.numpy as jnp
from jax import lax
from jax.experimental import pallas as pl
from jax.experimental.pallas import tpu as pltpu
from jax.experimental.pallas import tpu_sc as plsc

_B = 16384
_D = 128
_K = 38
_ROWS = 4096
_SCB = 4096
_NW = 32
_RPW = _SCB // _NW
_UNROLL = 2


def _mlp(xt, w1_ref, b1_ref, w2_ref, b2_ref, w3_ref, b3_ref):
    h = jnp.maximum(
        jnp.dot(w1_ref[:], xt, preferred_element_type=jnp.float32) + b1_ref[:], 0.0
    )
    h = jnp.dot(w2_ref[:], h, preferred_element_type=jnp.float32) + b2_ref[:]
    t = jnp.dot(w3_ref[:], h, preferred_element_type=jnp.float32) + b3_ref[:]
    return 1.0 / (1.0 + jnp.exp(-t))


def _mlp_body(feat_ref, w1_ref, b1_ref, w2_ref, b2_ref, w3_ref, b3_ref, out_ref):
    xt = feat_ref[:].T
    out_ref[:] = _mlp(xt, w1_ref, b1_ref, w2_ref, b2_ref, w3_ref, b3_ref).T


def _fused_body(feat_ref, w1_ref, b1_ref, w2_ref, b2_ref, w3_ref, b3_ref, out_ref):
    xt = feat_ref[:].T
    x2 = _mlp(xt, w1_ref, b1_ref, w2_ref, b2_ref, w3_ref, b3_ref)
    xi = lax.bitcast_convert_type(x2, jnp.int32)
    lo = jnp.zeros((1, xt.shape[1]), dtype=jnp.int32)
    hi = jnp.full((1, xt.shape[1]), jnp.int32(0x3F800000))
    for _ in range(30):
        mid = lax.shift_right_logical(lo + hi, 1)
        c = jnp.sum((xi <= mid).astype(jnp.int32), axis=0, keepdims=True)
        geq = c >= _K
        hi = jnp.where(geq, mid, hi)
        lo = jnp.where(geq, lo, mid + 1)
    out_ref[:] = jnp.where(xi > lo, x2, 0.0).T


def _tc_call(body, nrows, feature, W1, bb1, W2, bb2, W3, bb3):
    rows = min(_ROWS, nrows)
    row_spec = pl.BlockSpec((rows, _D), lambda i: (i, 0))
    w_spec = pl.BlockSpec((_D, _D), lambda i: (0, 0))
    b_spec = pl.BlockSpec((_D, 1), lambda i: (0, 0))
    return pl.pallas_call(
        body,
        grid=(nrows // rows,),
        in_specs=[row_spec, w_spec, b_spec, w_spec, b_spec, w_spec, b_spec],
        out_specs=row_spec,
        out_shape=jax.ShapeDtypeStruct((nrows, _D), jnp.float32),
    )(feature, W1, bb1, W2, bb2, W3, bb3)


def _sc_mask_body(x2_hbm, out_hbm, buf, sem):
    wid = lax.axis_index("s") * 2 + lax.axis_index("c")
    base = wid * _RPW
    pltpu.async_copy(x2_hbm.at[pl.ds(base, _RPW)], buf, sem).wait()

    iota = lax.iota(jnp.int32, 16)
    perms = [iota ^ (1 << p) for p in range(4)]
    _dn = lax.GatherDimensionNumbers(
        offset_dims=(), collapsed_slice_dims=(0,), start_index_map=(0,)
    )

    def _lane_sum(c):
        for p in perms:
            c = c + lax.gather(
                c, p[:, None], _dn, (1,),
                mode=lax.GatherScatterMode.PROMISE_IN_BOUNDS,
            )
        return c

    def row_body(g, carry):
        for u in range(_UNROLL):
            r = g * _UNROLL + u
            vals = [buf[r, pl.ds(16 * j, 16)] for j in range(8)]
            xis = [lax.bitcast_convert_type(v, jnp.int32) for v in vals]
            lo = jnp.zeros((16,), jnp.int32)
            hi = jnp.full((16,), 0x3F800000, jnp.int32)
            for _ in range(30):
                mid = lax.shift_right_logical(lo + hi, 1)
                acc = jnp.where(xis[0] <= mid, 1, 0)
                for j in range(1, 8):
                    acc = acc + jnp.where(xis[j] <= mid, 1, 0)
                c = _lane_sum(acc)
                geq = c >= _K
                hi = jnp.where(geq, mid, hi)
                lo = jnp.where(geq, lo, mid + 1)
            for j in range(8):
                buf[r, pl.ds(16 * j, 16)] = jnp.where(xis[j] > lo, vals[j], 0.0)
        return carry

    lax.fori_loop(0, _RPW // _UNROLL, row_body, jnp.int32(0))
    pltpu.async_copy(buf, out_hbm.at[pl.ds(base, _RPW)], sem).wait()


_sc_mask = functools.partial(
    pl.kernel,
    out_type=jax.ShapeDtypeStruct((_SCB, _D), jnp.float32),
    mesh=plsc.VectorSubcoreMesh(core_axis_name="c", subcore_axis_name="s"),
    scratch_types=[
        pltpu.VMEM((_RPW, _D), jnp.float32),
        pltpu.SemaphoreType.DMA,
    ],
)(_sc_mask_body)


@jax.jit
def kernel(feature, W1, b1, W2, b2, W3, b3):
    bb1 = b1.reshape(_D, 1)
    bb2 = b2.reshape(_D, 1)
    bb3 = b3.reshape(_D, 1)
    x2a = _tc_call(_mlp_body, _SCB, feature[:_SCB], W1, bb1, W2, bb2, W3, bb3)
    out_a = _sc_mask(x2a)
    out_b = _tc_call(
        _fused_body, _B - _SCB, feature[_SCB:], W1, bb1, W2, bb2, W3, bb3
    )
    return jnp.concatenate([out_a, out_b], axis=0)


# speedup vs baseline: 4.8562x; 1.1049x over previous
import functools

import jax
import jax.numpy as jnp
from jax import lax
from jax.experimental import pallas as pl
from jax.experimental.pallas import tpu as pltpu
from jax.experimental.pallas import tpu_sc as plsc

_B = 16384
_D = 128
_K = 38
_ROWS = 4096
_SCB = 2048
_NW = 32
_RPW = _SCB // _NW
_UNROLL = 2


def _mlp(xt, w1_ref, b1_ref, w2_ref, b2_ref, w3_ref, b3_ref):
    h = jnp.maximum(
        jnp.dot(w1_ref[:], xt, preferred_element_type=jnp.float32) + b1_ref[:], 0.0
    )
    h = jnp.dot(w2_ref[:], h, preferred_element_type=jnp.float32) + b2_ref[:]
    t = jnp.dot(w3_ref[:], h, preferred_element_type=jnp.float32) + b3_ref[:]
    return 1.0 / (1.0 + jnp.exp(-t))


def _mlp_body(feat_ref, w1_ref, b1_ref, w2_ref, b2_ref, w3_ref, b3_ref, out_ref):
    xt = feat_ref[:].T
    out_ref[:] = _mlp(xt, w1_ref, b1_ref, w2_ref, b2_ref, w3_ref, b3_ref).T


def _fused_body(feat_ref, w1_ref, b1_ref, w2_ref, b2_ref, w3_ref, b3_ref, out_ref):
    xt = feat_ref[:].T
    x2 = _mlp(xt, w1_ref, b1_ref, w2_ref, b2_ref, w3_ref, b3_ref)
    xi = lax.bitcast_convert_type(x2, jnp.int32)
    lo = jnp.zeros((1, xt.shape[1]), dtype=jnp.int32)
    hi = jnp.full((1, xt.shape[1]), jnp.int32(0x3F800000))
    for _ in range(30):
        mid = lax.shift_right_logical(lo + hi, 1)
        c = jnp.sum((xi <= mid).astype(jnp.int32), axis=0, keepdims=True)
        geq = c >= _K
        hi = jnp.where(geq, mid, hi)
        lo = jnp.where(geq, lo, mid + 1)
    out_ref[:] = jnp.where(xi > lo, x2, 0.0).T


def _tc_call(body, nrows, feature, W1, bb1, W2, bb2, W3, bb3):
    rows = min(_ROWS, nrows)
    row_spec = pl.BlockSpec((rows, _D), lambda i: (i, 0))
    w_spec = pl.BlockSpec((_D, _D), lambda i: (0, 0))
    b_spec = pl.BlockSpec((_D, 1), lambda i: (0, 0))
    return pl.pallas_call(
        body,
        grid=(nrows // rows,),
        in_specs=[row_spec, w_spec, b_spec, w_spec, b_spec, w_spec, b_spec],
        out_specs=row_spec,
        out_shape=jax.ShapeDtypeStruct((nrows, _D), jnp.float32),
    )(feature, W1, bb1, W2, bb2, W3, bb3)


def _sc_mask_body(x2_hbm, out_hbm, buf, sem):
    wid = lax.axis_index("s") * 2 + lax.axis_index("c")
    base = wid * _RPW
    pltpu.async_copy(x2_hbm.at[pl.ds(base, _RPW)], buf, sem).wait()

    iota = lax.iota(jnp.int32, 16)
    perms = [iota ^ (1 << p) for p in range(4)]
    _dn = lax.GatherDimensionNumbers(
        offset_dims=(), collapsed_slice_dims=(0,), start_index_map=(0,)
    )

    def _lane_sum(c):
        for p in perms:
            c = c + lax.gather(
                c, p[:, None], _dn, (1,),
                mode=lax.GatherScatterMode.PROMISE_IN_BOUNDS,
            )
        return c

    def row_body(g, carry):
        for u in range(_UNROLL):
            r = g * _UNROLL + u
            vals = [buf[r, pl.ds(16 * j, 16)] for j in range(8)]
            xis = [lax.bitcast_convert_type(v, jnp.int32) for v in vals]
            lo = jnp.zeros((16,), jnp.int32)
            hi = jnp.full((16,), 0x3F800000, jnp.int32)
            for _ in range(30):
                mid = lax.shift_right_logical(lo + hi, 1)
                acc = jnp.where(xis[0] <= mid, 1, 0)
                for j in range(1, 8):
                    acc = acc + jnp.where(xis[j] <= mid, 1, 0)
                c = _lane_sum(acc)
                geq = c >= _K
                hi = jnp.where(geq, mid, hi)
                lo = jnp.where(geq, lo, mid + 1)
            for j in range(8):
                buf[r, pl.ds(16 * j, 16)] = jnp.where(xis[j] > lo, vals[j], 0.0)
        return carry

    lax.fori_loop(0, _RPW // _UNROLL, row_body, jnp.int32(0))
    pltpu.async_copy(buf, out_hbm.at[pl.ds(base, _RPW)], sem).wait()


_sc_mask = functools.partial(
    pl.kernel,
    out_type=jax.ShapeDtypeStruct((_SCB, _D), jnp.float32),
    mesh=plsc.VectorSubcoreMesh(core_axis_name="c", subcore_axis_name="s"),
    scratch_types=[
        pltpu.VMEM((_RPW, _D), jnp.float32),
        pltpu.SemaphoreType.DMA,
    ],
)(_sc_mask_body)


@jax.jit
def kernel(feature, W1, b1, W2, b2, W3, b3):
    bb1 = b1.reshape(_D, 1)
    bb2 = b2.reshape(_D, 1)
    bb3 = b3.reshape(_D, 1)
    x2a = _tc_call(_mlp_body, _SCB, feature[:_SCB], W1, bb1, W2, bb2, W3, bb3)
    out_a = _sc_mask(x2a)
    out_b = _tc_call(
        _fused_body, _B - _SCB, feature[_SCB:], W1, bb1, W2, bb2, W3, bb3
    )
    return jnp.concatenate([out_a, out_b], axis=0)
